# baseline (speedup 1.0000x reference)
import functools

import jax
import jax.numpy as jnp
from jax import lax
from jax.experimental import pallas as pl
from jax.experimental.pallas import tpu as pltpu
from jax.experimental.pallas import tpu_sc as plsc

_ROWS = 16384
_D = 128
_KEEP = 64

_info = plsc.get_sparse_core_info()
_NC = _info.num_cores
_NS = _info.num_subcores
_NW = _NC * _NS
_ROWS_PER_W = _ROWS // _NW

_mesh = plsc.VectorSubcoreMesh(core_axis_name="c", subcore_axis_name="s")


@functools.partial(
    pl.kernel,
    mesh=_mesh,
    out_type=jax.ShapeDtypeStruct((_ROWS, _KEEP), jnp.float32),
    compiler_params=pltpu.CompilerParams(use_tc_tiling_on_sc=False),
)
def _mask_copy(x_hbm, out_hbm):
    wid = lax.axis_index("s") * _NC + lax.axis_index("c")
    base = wid * _ROWS_PER_W
    pltpu.sync_copy(
        x_hbm.at[pl.ds(base, _ROWS_PER_W), pl.ds(0, _KEEP)],
        out_hbm.at[pl.ds(base, _ROWS_PER_W)],
    )


def kernel(inputs):
    return _mask_copy(inputs)


# speedup vs baseline: 1.0009x; 1.0009x over previous
import functools

import jax
import jax.numpy as jnp
from jax import lax
from jax.experimental import pallas as pl
from jax.experimental.pallas import tpu as pltpu
from jax.experimental.pallas import tpu_sc as plsc

_ROWS = 16384
_D = 128
_KEEP = 64

_info = plsc.get_sparse_core_info()
_NC = _info.num_cores
_NS = _info.num_subcores
_NW = _NC * _NS
_ROWS_PER_W = _ROWS // _NW

_mesh = plsc.VectorSubcoreMesh(core_axis_name="c", subcore_axis_name="s")


_NCHUNK = 8
_CHUNK = _ROWS_PER_W // _NCHUNK


@functools.partial(
    pl.kernel,
    mesh=_mesh,
    out_type=jax.ShapeDtypeStruct((_ROWS, _KEEP), jnp.float32),
    scratch_types=[pltpu.SemaphoreType.DMA],
    compiler_params=pltpu.CompilerParams(use_tc_tiling_on_sc=False),
)
def _mask_copy(x_hbm, out_hbm, sem):
    wid = lax.axis_index("s") * _NC + lax.axis_index("c")
    base = wid * _ROWS_PER_W
    copies = []
    for k in range(_NCHUNK):
        lo = base + k * _CHUNK
        copies.append(
            pltpu.async_copy(
                x_hbm.at[pl.ds(lo, _CHUNK), pl.ds(0, _KEEP)],
                out_hbm.at[pl.ds(lo, _CHUNK)],
                sem,
            )
        )
    for cp in copies:
        cp.wait()


def kernel(inputs):
    return _mask_copy(inputs)


# speedup vs baseline: 4.3715x; 4.3674x over previous
import functools

import jax
import jax.numpy as jnp
from jax import lax
from jax.experimental import pallas as pl
from jax.experimental.pallas import tpu as pltpu
from jax.experimental.pallas import tpu_sc as plsc

_ROWS = 16384
_D = 128
_KEEP = 64

_info = plsc.get_sparse_core_info()
_NC = _info.num_cores
_NS = _info.num_subcores
_NW = _NC * _NS
_ROWS_PER_W = _ROWS // _NW

_mesh = plsc.VectorSubcoreMesh(core_axis_name="c", subcore_axis_name="s")


@functools.partial(
    pl.kernel,
    mesh=_mesh,
    out_type=jax.ShapeDtypeStruct((_ROWS, _KEEP), jnp.float32),
    scratch_types=[
        pltpu.VMEM((_ROWS_PER_W, _KEEP), jnp.float32),
        pltpu.SemaphoreType.DMA,
        pltpu.SemaphoreType.DMA,
    ],
    compiler_params=pltpu.CompilerParams(use_tc_tiling_on_sc=False),
)
def _mask_copy(x_hbm, out_hbm, buf, in_sem, out_sem):
    wid = lax.axis_index("s") * _NC + lax.axis_index("c")
    base = wid * _ROWS_PER_W
    pltpu.async_copy(
        x_hbm.at[pl.ds(base, _ROWS_PER_W), pl.ds(0, _KEEP)], buf, in_sem
    ).wait()
    pltpu.async_copy(
        buf, out_hbm.at[pl.ds(base, _ROWS_PER_W)], out_sem
    ).wait()


def kernel(inputs):
    return _mask_copy(inputs)


# speedup vs baseline: 4.3719x; 1.0001x over previous
import functools

import jax
import jax.numpy as jnp
from jax import lax
from jax.experimental import pallas as pl
from jax.experimental.pallas import tpu as pltpu
from jax.experimental.pallas import tpu_sc as plsc

_ROWS = 16384
_D = 128
_KEEP = 64

_info = plsc.get_sparse_core_info()
_NC = _info.num_cores
_NS = _info.num_subcores
_NW = _NC * _NS
_ROWS_PER_W = _ROWS // _NW

_mesh = plsc.VectorSubcoreMesh(core_axis_name="c", subcore_axis_name="s")


_NCHUNK = 4
_CHUNK = _ROWS_PER_W // _NCHUNK


@functools.partial(
    pl.kernel,
    mesh=_mesh,
    out_type=jax.ShapeDtypeStruct((_ROWS, _KEEP), jnp.float32),
    scratch_types=[
        pltpu.VMEM((_ROWS_PER_W, _KEEP), jnp.float32),
        [pltpu.SemaphoreType.DMA] * _NCHUNK,
        pltpu.SemaphoreType.DMA,
    ],
    compiler_params=pltpu.CompilerParams(use_tc_tiling_on_sc=False),
)
def _mask_copy(x_hbm, out_hbm, buf, in_sems, out_sem):
    wid = lax.axis_index("s") * _NC + lax.axis_index("c")
    base = wid * _ROWS_PER_W
    ins = []
    for k in range(_NCHUNK):
        lo = base + k * _CHUNK
        ins.append(
            pltpu.async_copy(
                x_hbm.at[pl.ds(lo, _CHUNK), pl.ds(0, _KEEP)],
                buf.at[pl.ds(k * _CHUNK, _CHUNK)],
                in_sems[k],
            )
        )
    outs = []
    for k in range(_NCHUNK):
        ins[k].wait()
        outs.append(
            pltpu.async_copy(
                buf.at[pl.ds(k * _CHUNK, _CHUNK)],
                out_hbm.at[pl.ds(base + k * _CHUNK, _CHUNK)],
                out_sem,
            )
        )
    for cp in outs:
        cp.wait()


def kernel(inputs):
    return _mask_copy(inputs)


# speedup vs baseline: 4.3751x; 1.0007x over previous
import functools

import jax
import jax.numpy as jnp
from jax import lax
from jax.experimental import pallas as pl
from jax.experimental.pallas import tpu as pltpu
from jax.experimental.pallas import tpu_sc as plsc

_ROWS = 16384
_D = 128
_KEEP = 64

_info = plsc.get_sparse_core_info()
_NC = _info.num_cores
_NS = _info.num_subcores
_NW = _NC * _NS
_ROWS_PER_W = _ROWS // _NW

_mesh = plsc.VectorSubcoreMesh(core_axis_name="c", subcore_axis_name="s")


_NCHUNK = 4
_CHUNK = _ROWS_PER_W // _NCHUNK


@functools.partial(
    pl.kernel,
    mesh=_mesh,
    out_type=jax.ShapeDtypeStruct((_ROWS, _KEEP), jnp.float32),
    scratch_types=[
        pltpu.VMEM((_ROWS_PER_W, _KEEP), jnp.float32),
        [pltpu.SemaphoreType.DMA] * _NCHUNK,
        pltpu.SemaphoreType.DMA,
    ],
    compiler_params=pltpu.CompilerParams(
        use_tc_tiling_on_sc=False,
        disable_bounds_checks=True,
        disable_semaphore_checks=True,
    ),
)
def _mask_copy(x_hbm, out_hbm, buf, in_sems, out_sem):
    wid = lax.axis_index("s") * _NC + lax.axis_index("c")
    base = wid * _ROWS_PER_W
    ins = []
    for k in range(_NCHUNK):
        lo = base + k * _CHUNK
        ins.append(
            pltpu.async_copy(
                x_hbm.at[pl.ds(lo, _CHUNK), pl.ds(0, _KEEP)],
                buf.at[pl.ds(k * _CHUNK, _CHUNK)],
                in_sems[k],
            )
        )
    outs = []
    for k in range(_NCHUNK):
        ins[k].wait()
        outs.append(
            pltpu.async_copy(
                buf.at[pl.ds(k * _CHUNK, _CHUNK)],
                out_hbm.at[pl.ds(base + k * _CHUNK, _CHUNK)],
                out_sem,
            )
        )
    for cp in outs:
        cp.wait()


def kernel(inputs):
    return _mask_copy(inputs)
